# baseline (device time: 126268 ns/iter reference)
import jax
import jax.numpy as jnp
from jax import lax
from jax.experimental import pallas as pl
from jax.experimental.pallas import tpu as pltpu

N_DEV = 16
H = 16
DH = 128
DR = 32
DC = 128
S = 1024
D = 2048


def _ring_pos(mesh):
    q = mesh % 4
    z = mesh // 4
    return jnp.where(
        q == 0, z,
        jnp.where(q == 1, 7 - z, jnp.where(q == 2, 8 + z, 15 - z)))


def _mesh_of(r):
    r = r % N_DEV
    q = r // 4
    z = jnp.where(
        q == 0, r,
        jnp.where(q == 1, 7 - r, jnp.where(q == 2, r - 8, 15 - r)))
    return 4 * z + q


def _mla_body(xf_ref, wdkv_ref, wuk_ref, wuv_ref, wq_ref, wqr_ref, wkr_ref,
              wo_ref, out_ref,
              c_all, wstage, wrecv, kvacc, q_s, qr_s, kr_s, oslots,
              a2a_send_sems, a2a_recv_sems,
              c_cw_send, c_cw_recv, c_ccw_send, c_ccw_recv,
              o_cw_send, o_cw_recv, o_ccw_send, o_ccw_recv):
    my_mesh = lax.axis_index("i")
    my = _ring_pos(my_mesh)
    left = _mesh_of(my - 1)
    right = _mesh_of(my + 1)
    bf = jnp.bfloat16

    barrier = pltpu.get_barrier_semaphore()
    for off in range(1, N_DEV):
        pl.semaphore_signal(
            barrier, inc=1, device_id=((my_mesh + off) % N_DEV,),
            device_id_type=pl.DeviceIdType.MESH,
        )
    pl.semaphore_wait(barrier, N_DEV - 1)

    for d in range(N_DEV):
        wstage[d, 0] = wuk_ref[:, d * DH:(d + 1) * DH]
        wstage[d, 1] = wuv_ref[:, d * DH:(d + 1) * DH]
    wrecv[my_mesh] = wstage[my_mesh]
    a2a = []
    for off in range(1, N_DEV):
        dst = (my_mesh + off) % N_DEV
        r = pltpu.make_async_remote_copy(
            src_ref=wstage.at[dst],
            dst_ref=wrecv.at[my_mesh],
            send_sem=a2a_send_sems.at[dst],
            recv_sem=a2a_recv_sems.at[my_mesh],
            device_id=(dst,),
            device_id_type=pl.DeviceIdType.MESH,
        )
        r.start()
        a2a.append(r)

    c_all[my_mesh] = jnp.dot(
        xf_ref[...], wdkv_ref[...],
        preferred_element_type=jnp.float32).astype(bf)
    kvacc[0] = jnp.dot(c_all[my_mesh], wrecv[my_mesh, 0],
                       preferred_element_type=jnp.float32)
    kvacc[1] = jnp.dot(c_all[my_mesh], wrecv[my_mesh, 1],
                       preferred_element_type=jnp.float32)

    def absorb(k):
        w_arrived = pltpu.make_async_remote_copy(
            src_ref=wstage.at[k],
            dst_ref=wrecv.at[k],
            send_sem=a2a_send_sems.at[k],
            recv_sem=a2a_recv_sems.at[k],
            device_id=(k,),
            device_id_type=pl.DeviceIdType.MESH,
        )
        w_arrived.wait_recv()
        kvacc[0] = kvacc[0] + jnp.dot(
            c_all[k], wrecv[k, 0], preferred_element_type=jnp.float32)
        kvacc[1] = kvacc[1] + jnp.dot(
            c_all[k], wrecv[k, 1], preferred_element_type=jnp.float32)

    for s in range(8):
        k_cw_send = _mesh_of(my - s)
        k_cw_recv = _mesh_of(my - s - 1)
        cw = pltpu.make_async_remote_copy(
            src_ref=c_all.at[k_cw_send],
            dst_ref=c_all.at[k_cw_send],
            send_sem=c_cw_send.at[s],
            recv_sem=c_cw_recv.at[s],
            device_id=(right,),
            device_id_type=pl.DeviceIdType.MESH,
        )
        cw.start()
        if s < 7:
            k_ccw_send = _mesh_of(my + s)
            k_ccw_recv = _mesh_of(my + s + 1)
            ccw = pltpu.make_async_remote_copy(
                src_ref=c_all.at[k_ccw_send],
                dst_ref=c_all.at[k_ccw_send],
                send_sem=c_ccw_send.at[s],
                recv_sem=c_ccw_recv.at[s],
                device_id=(left,),
                device_id_type=pl.DeviceIdType.MESH,
            )
            ccw.start()
        if s >= 1:
            absorb(_mesh_of(my - s))
            absorb(_mesh_of(my + s))
        if s == 0:
            q_s[...] = jnp.dot(
                xf_ref[...], wq_ref[...],
                preferred_element_type=jnp.float32).astype(bf)
        if s == 1:
            qr_s[...] = jnp.dot(
                xf_ref[...], wqr_ref[...],
                preferred_element_type=jnp.float32).astype(bf)
            kr_s[...] = jnp.dot(
                xf_ref[...], wkr_ref[...],
                preferred_element_type=jnp.float32).astype(bf)
        cw_recv = pltpu.make_async_remote_copy(
            src_ref=c_all.at[k_cw_recv],
            dst_ref=c_all.at[k_cw_recv],
            send_sem=c_cw_send.at[s],
            recv_sem=c_cw_recv.at[s],
            device_id=(left,),
            device_id_type=pl.DeviceIdType.MESH,
        )
        cw_recv.wait_recv()
        cw.wait_send()
        if s < 7:
            ccw_recv = pltpu.make_async_remote_copy(
                src_ref=c_all.at[k_ccw_recv],
                dst_ref=c_all.at[k_ccw_recv],
                send_sem=c_ccw_send.at[s],
                recv_sem=c_ccw_recv.at[s],
                device_id=(right,),
                device_id_type=pl.DeviceIdType.MESH,
            )
            ccw_recv.wait_recv()
            ccw.wait_send()
    absorb(_mesh_of(my - 8))

    scale = (DH + DR) ** -0.5
    k_bf = kvacc[0].astype(bf)
    v_bf = kvacc[1].astype(bf)
    contract_last = (((1,), (1,)), ((), ()))
    scores = (
        lax.dot_general(q_s[...], k_bf, contract_last,
                        preferred_element_type=jnp.float32)
        + lax.dot_general(qr_s[...], kr_s[...], contract_last,
                          preferred_element_type=jnp.float32)
    ) * scale
    m = jnp.max(scores, axis=-1, keepdims=True)
    p = jnp.exp(scores - m)
    p = p / jnp.sum(p, axis=-1, keepdims=True)
    oslots[my_mesh] = jnp.dot(
        p.astype(bf), v_bf, preferred_element_type=jnp.float32).astype(bf)

    def head_out(k):
        return jnp.dot(
            oslots[k], wo_ref[pl.ds(k * DH, DH), :],
            preferred_element_type=jnp.float32,
        )

    for s in range(8):
        k_cw_send = _mesh_of(my - s)
        k_cw_recv = _mesh_of(my - s - 1)
        cw = pltpu.make_async_remote_copy(
            src_ref=oslots.at[k_cw_send],
            dst_ref=oslots.at[k_cw_send],
            send_sem=o_cw_send.at[s],
            recv_sem=o_cw_recv.at[s],
            device_id=(right,),
            device_id_type=pl.DeviceIdType.MESH,
        )
        cw.start()
        if s < 7:
            k_ccw_send = _mesh_of(my + s)
            k_ccw_recv = _mesh_of(my + s + 1)
            ccw = pltpu.make_async_remote_copy(
                src_ref=oslots.at[k_ccw_send],
                dst_ref=oslots.at[k_ccw_send],
                send_sem=o_ccw_send.at[s],
                recv_sem=o_ccw_recv.at[s],
                device_id=(left,),
                device_id_type=pl.DeviceIdType.MESH,
            )
            ccw.start()
        if s == 0:
            out_ref[...] = head_out(my_mesh)
        else:
            out_ref[...] = out_ref[...] + head_out(_mesh_of(my - s))
            out_ref[...] = out_ref[...] + head_out(_mesh_of(my + s))
        cw_recv = pltpu.make_async_remote_copy(
            src_ref=oslots.at[k_cw_recv],
            dst_ref=oslots.at[k_cw_recv],
            send_sem=o_cw_send.at[s],
            recv_sem=o_cw_recv.at[s],
            device_id=(left,),
            device_id_type=pl.DeviceIdType.MESH,
        )
        cw_recv.wait_recv()
        cw.wait_send()
        if s < 7:
            ccw_recv = pltpu.make_async_remote_copy(
                src_ref=oslots.at[k_ccw_recv],
                dst_ref=oslots.at[k_ccw_recv],
                send_sem=o_ccw_send.at[s],
                recv_sem=o_ccw_recv.at[s],
                device_id=(right,),
                device_id_type=pl.DeviceIdType.MESH,
            )
            ccw_recv.wait_recv()
            ccw.wait_send()
    out_ref[...] = out_ref[...] + head_out(_mesh_of(my - 8))
    for r in a2a:
        r.wait_send()


def kernel(x, Wdkv, Wuk, Wuv, Wq, Wqr, Wkr, Wo):
    my = lax.axis_index("i")
    bf = jnp.bfloat16
    xf = x[0].astype(bf)
    Wq_h = lax.dynamic_slice(Wq, (0, my * DH), (D, DH)).astype(bf)
    Wqr_h = lax.dynamic_slice(Wqr, (0, my * DR), (D, DR)).astype(bf)

    out = pl.pallas_call(
        _mla_body,
        out_shape=jax.ShapeDtypeStruct((S, D), jnp.float32),
        in_specs=[pl.BlockSpec(memory_space=pltpu.VMEM)] * 8,
        out_specs=pl.BlockSpec(memory_space=pltpu.VMEM),
        scratch_shapes=[
            pltpu.VMEM((N_DEV, S, DC), jnp.bfloat16),
            pltpu.VMEM((N_DEV, 2, DC, DH), jnp.bfloat16),
            pltpu.VMEM((N_DEV, 2, DC, DH), jnp.bfloat16),
            pltpu.VMEM((2, S, DH), jnp.float32),
            pltpu.VMEM((S, DH), jnp.bfloat16),
            pltpu.VMEM((S, DR), jnp.bfloat16),
            pltpu.VMEM((S, DR), jnp.bfloat16),
            pltpu.VMEM((N_DEV, S, DH), jnp.bfloat16),
            pltpu.SemaphoreType.DMA((N_DEV,)),
            pltpu.SemaphoreType.DMA((N_DEV,)),
            pltpu.SemaphoreType.DMA((8,)),
            pltpu.SemaphoreType.DMA((8,)),
            pltpu.SemaphoreType.DMA((7,)),
            pltpu.SemaphoreType.DMA((7,)),
            pltpu.SemaphoreType.DMA((8,)),
            pltpu.SemaphoreType.DMA((8,)),
            pltpu.SemaphoreType.DMA((7,)),
            pltpu.SemaphoreType.DMA((7,)),
        ],
        compiler_params=pltpu.CompilerParams(collective_id=0),
    )(xf, Wdkv.astype(bf), Wuk.astype(bf), Wuv.astype(bf),
      Wq_h, Wqr_h, Wkr.astype(bf), Wo.astype(bf))
    return out[None]


# device time: 115224 ns/iter; 1.0958x vs baseline; 1.0958x over previous
import jax
import jax.numpy as jnp
from jax import lax
from jax.experimental import pallas as pl
from jax.experimental.pallas import tpu as pltpu

N_DEV = 16
H = 16
DH = 128
DR = 32
DC = 128
S = 1024
D = 2048


def _ring_pos(mesh):
    q = mesh % 4
    z = mesh // 4
    return jnp.where(
        q == 0, z,
        jnp.where(q == 1, 7 - z, jnp.where(q == 2, 8 + z, 15 - z)))


def _mesh_of(r):
    r = r % N_DEV
    q = r // 4
    z = jnp.where(
        q == 0, r,
        jnp.where(q == 1, 7 - r, jnp.where(q == 2, r - 8, 15 - r)))
    return 4 * z + q


def _neighbor_barrier(left, right):
    barrier = pltpu.get_barrier_semaphore()
    for nbr in (left, right):
        pl.semaphore_signal(
            barrier, inc=1, device_id=(nbr,),
            device_id_type=pl.DeviceIdType.MESH,
        )
    pl.semaphore_wait(barrier, 2)


def _prep_body(xf_ref, wdkv_ref, wuk_ref, wuv_ref, wq_ref, wqr_ref, wkr_ref,
               kv_ref, q_ref, qr_ref, kr_ref,
               c_all, wstage, wrecv, kvacc,
               a2a_send_sems, a2a_recv_sems,
               cw_send_sems, cw_recv_sems, ccw_send_sems, ccw_recv_sems):
    my_mesh = lax.axis_index("i")
    my = _ring_pos(my_mesh)
    left = _mesh_of(my - 1)
    right = _mesh_of(my + 1)
    bf = jnp.bfloat16

    barrier = pltpu.get_barrier_semaphore()
    for off in range(1, N_DEV):
        pl.semaphore_signal(
            barrier, inc=1, device_id=((my_mesh + off) % N_DEV,),
            device_id_type=pl.DeviceIdType.MESH,
        )
    pl.semaphore_wait(barrier, N_DEV - 1)

    for d in range(N_DEV):
        wstage[d, 0] = wuk_ref[:, d * DH:(d + 1) * DH]
        wstage[d, 1] = wuv_ref[:, d * DH:(d + 1) * DH]
    wrecv[my_mesh] = wstage[my_mesh]
    a2a = []
    for off in range(1, N_DEV):
        dst = (my_mesh + off) % N_DEV
        r = pltpu.make_async_remote_copy(
            src_ref=wstage.at[dst],
            dst_ref=wrecv.at[my_mesh],
            send_sem=a2a_send_sems.at[dst],
            recv_sem=a2a_recv_sems.at[my_mesh],
            device_id=(dst,),
            device_id_type=pl.DeviceIdType.MESH,
        )
        r.start()
        a2a.append(r)

    c_all[my_mesh] = jnp.dot(
        xf_ref[...], wdkv_ref[...],
        preferred_element_type=jnp.float32).astype(bf)
    kvacc[0] = jnp.dot(c_all[my_mesh], wrecv[my_mesh, 0],
                       preferred_element_type=jnp.float32)
    kvacc[1] = jnp.dot(c_all[my_mesh], wrecv[my_mesh, 1],
                       preferred_element_type=jnp.float32)

    def absorb(k):
        w_arrived = pltpu.make_async_remote_copy(
            src_ref=wstage.at[k],
            dst_ref=wrecv.at[k],
            send_sem=a2a_send_sems.at[k],
            recv_sem=a2a_recv_sems.at[k],
            device_id=(k,),
            device_id_type=pl.DeviceIdType.MESH,
        )
        w_arrived.wait_recv()
        kvacc[0] = kvacc[0] + jnp.dot(
            c_all[k], wrecv[k, 0], preferred_element_type=jnp.float32)
        kvacc[1] = kvacc[1] + jnp.dot(
            c_all[k], wrecv[k, 1], preferred_element_type=jnp.float32)

    for s in range(8):
        k_cw_send = _mesh_of(my - s)
        k_cw_recv = _mesh_of(my - s - 1)
        cw = pltpu.make_async_remote_copy(
            src_ref=c_all.at[k_cw_send],
            dst_ref=c_all.at[k_cw_send],
            send_sem=cw_send_sems.at[s],
            recv_sem=cw_recv_sems.at[s],
            device_id=(right,),
            device_id_type=pl.DeviceIdType.MESH,
        )
        cw.start()
        if s < 7:
            k_ccw_send = _mesh_of(my + s)
            k_ccw_recv = _mesh_of(my + s + 1)
            ccw = pltpu.make_async_remote_copy(
                src_ref=c_all.at[k_ccw_send],
                dst_ref=c_all.at[k_ccw_send],
                send_sem=ccw_send_sems.at[s],
                recv_sem=ccw_recv_sems.at[s],
                device_id=(left,),
                device_id_type=pl.DeviceIdType.MESH,
            )
            ccw.start()
        if s >= 1:
            absorb(_mesh_of(my - s))
            absorb(_mesh_of(my + s))
        if s == 0:
            q_ref[...] = jnp.dot(
                xf_ref[...], wq_ref[...],
                preferred_element_type=jnp.float32).astype(bf)
        if s == 1:
            qr_ref[...] = jnp.dot(
                xf_ref[...], wqr_ref[...],
                preferred_element_type=jnp.float32).astype(bf)
            kr_ref[...] = jnp.dot(
                xf_ref[...], wkr_ref[...],
                preferred_element_type=jnp.float32).astype(bf)
        cw_recv = pltpu.make_async_remote_copy(
            src_ref=c_all.at[k_cw_recv],
            dst_ref=c_all.at[k_cw_recv],
            send_sem=cw_send_sems.at[s],
            recv_sem=cw_recv_sems.at[s],
            device_id=(left,),
            device_id_type=pl.DeviceIdType.MESH,
        )
        cw_recv.wait_recv()
        cw.wait_send()
        if s < 7:
            ccw_recv = pltpu.make_async_remote_copy(
                src_ref=c_all.at[k_ccw_recv],
                dst_ref=c_all.at[k_ccw_recv],
                send_sem=ccw_send_sems.at[s],
                recv_sem=ccw_recv_sems.at[s],
                device_id=(right,),
                device_id_type=pl.DeviceIdType.MESH,
            )
            ccw_recv.wait_recv()
            ccw.wait_send()
    absorb(_mesh_of(my - 8))
    kv_ref[0] = kvacc[0].astype(bf)
    kv_ref[1] = kvacc[1].astype(bf)
    for r in a2a:
        r.wait_send()


def _prepare(xf, wdkv, wuk, wuv, wq_h, wqr_h, wkr):
    return pl.pallas_call(
        _prep_body,
        out_shape=(
            jax.ShapeDtypeStruct((2, S, DH), jnp.bfloat16),
            jax.ShapeDtypeStruct((S, DH), jnp.bfloat16),
            jax.ShapeDtypeStruct((S, DR), jnp.bfloat16),
            jax.ShapeDtypeStruct((S, DR), jnp.bfloat16),
        ),
        in_specs=[pl.BlockSpec(memory_space=pltpu.VMEM)] * 7,
        out_specs=(pl.BlockSpec(memory_space=pltpu.VMEM),) * 4,
        scratch_shapes=[
            pltpu.VMEM((N_DEV, S, DC), jnp.bfloat16),
            pltpu.VMEM((N_DEV, 2, DC, DH), jnp.bfloat16),
            pltpu.VMEM((N_DEV, 2, DC, DH), jnp.bfloat16),
            pltpu.VMEM((2, S, DH), jnp.float32),
            pltpu.SemaphoreType.DMA((N_DEV,)),
            pltpu.SemaphoreType.DMA((N_DEV,)),
            pltpu.SemaphoreType.DMA((8,)),
            pltpu.SemaphoreType.DMA((8,)),
            pltpu.SemaphoreType.DMA((7,)),
            pltpu.SemaphoreType.DMA((7,)),
        ],
        compiler_params=pltpu.CompilerParams(collective_id=0),
    )(xf, wdkv, wuk, wuv, wq_h, wqr_h, wkr)


def _ag_body(o_ref, wo_ref, out_ref, slots,
             cw_send_sems, cw_recv_sems, ccw_send_sems, ccw_recv_sems):
    my_mesh = lax.axis_index("i")
    my = _ring_pos(my_mesh)
    left = _mesh_of(my - 1)
    right = _mesh_of(my + 1)
    _neighbor_barrier(left, right)

    slots[my_mesh] = o_ref[...]

    def head_out(k):
        return jnp.dot(
            slots[k], wo_ref[pl.ds(k * DH, DH), :],
            preferred_element_type=jnp.float32,
        )

    for s in range(8):
        k_cw_send = _mesh_of(my - s)
        k_cw_recv = _mesh_of(my - s - 1)
        cw = pltpu.make_async_remote_copy(
            src_ref=slots.at[k_cw_send],
            dst_ref=slots.at[k_cw_send],
            send_sem=cw_send_sems.at[s],
            recv_sem=cw_recv_sems.at[s],
            device_id=(right,),
            device_id_type=pl.DeviceIdType.MESH,
        )
        cw.start()
        if s < 7:
            k_ccw_send = _mesh_of(my + s)
            k_ccw_recv = _mesh_of(my + s + 1)
            ccw = pltpu.make_async_remote_copy(
                src_ref=slots.at[k_ccw_send],
                dst_ref=slots.at[k_ccw_send],
                send_sem=ccw_send_sems.at[s],
                recv_sem=ccw_recv_sems.at[s],
                device_id=(left,),
                device_id_type=pl.DeviceIdType.MESH,
            )
            ccw.start()
        if s == 0:
            out_ref[...] = head_out(my_mesh)
        else:
            out_ref[...] = out_ref[...] + head_out(_mesh_of(my - s))
            out_ref[...] = out_ref[...] + head_out(_mesh_of(my + s))
        cw_recv = pltpu.make_async_remote_copy(
            src_ref=slots.at[k_cw_recv],
            dst_ref=slots.at[k_cw_recv],
            send_sem=cw_send_sems.at[s],
            recv_sem=cw_recv_sems.at[s],
            device_id=(left,),
            device_id_type=pl.DeviceIdType.MESH,
        )
        cw_recv.wait_recv()
        cw.wait_send()
        if s < 7:
            ccw_recv = pltpu.make_async_remote_copy(
                src_ref=slots.at[k_ccw_recv],
                dst_ref=slots.at[k_ccw_recv],
                send_sem=ccw_send_sems.at[s],
                recv_sem=ccw_recv_sems.at[s],
                device_id=(right,),
                device_id_type=pl.DeviceIdType.MESH,
            )
            ccw_recv.wait_recv()
            ccw.wait_send()
    out_ref[...] = out_ref[...] + head_out(_mesh_of(my - 8))


def _all_gather_matmul(o_h, wo):
    return pl.pallas_call(
        _ag_body,
        out_shape=jax.ShapeDtypeStruct((S, D), jnp.float32),
        in_specs=[
            pl.BlockSpec(memory_space=pltpu.VMEM),
            pl.BlockSpec(memory_space=pltpu.VMEM),
        ],
        out_specs=pl.BlockSpec(memory_space=pltpu.VMEM),
        scratch_shapes=[
            pltpu.VMEM((N_DEV, S, DH), jnp.bfloat16),
            pltpu.SemaphoreType.DMA((8,)),
            pltpu.SemaphoreType.DMA((8,)),
            pltpu.SemaphoreType.DMA((7,)),
            pltpu.SemaphoreType.DMA((7,)),
        ],
        compiler_params=pltpu.CompilerParams(collective_id=1),
    )(o_h, wo)


def kernel(x, Wdkv, Wuk, Wuv, Wq, Wqr, Wkr, Wo):
    my = lax.axis_index("i")
    bf = jnp.bfloat16
    xf = x[0].astype(bf)
    Wq_h = lax.dynamic_slice(Wq, (0, my * DH), (D, DH)).astype(bf)
    Wqr_h = lax.dynamic_slice(Wqr, (0, my * DR), (D, DR)).astype(bf)
    kv, Q_h, Qr_h, Kr = _prepare(
        xf, Wdkv.astype(bf), Wuk.astype(bf), Wuv.astype(bf),
        Wq_h, Wqr_h, Wkr.astype(bf))
    K_h, V_h = kv[0], kv[1]

    scale = (DH + DR) ** -0.5
    scores = (
        jnp.dot(Q_h, K_h.T, preferred_element_type=jnp.float32)
        + jnp.dot(Qr_h, Kr.T, preferred_element_type=jnp.float32)
    ) * scale
    m = jnp.max(scores, axis=-1, keepdims=True)
    p = jnp.exp(scores - m)
    p = p / jnp.sum(p, axis=-1, keepdims=True)
    O_h = jnp.dot(p.astype(bf), V_h)

    out = _all_gather_matmul(O_h, Wo.astype(bf))
    return out[None]


# device time: 113713 ns/iter; 1.1104x vs baseline; 1.0133x over previous
import jax
import jax.numpy as jnp
from jax import lax
from jax.experimental import pallas as pl
from jax.experimental.pallas import tpu as pltpu

N_DEV = 16
H = 16
DH = 128
DR = 32
DC = 128
S = 1024
D = 2048


def _ring_pos(mesh):
    q = mesh % 4
    z = mesh // 4
    return jnp.where(
        q == 0, z,
        jnp.where(q == 1, 7 - z, jnp.where(q == 2, 8 + z, 15 - z)))


def _mesh_of(r):
    r = r % N_DEV
    q = r // 4
    z = jnp.where(
        q == 0, r,
        jnp.where(q == 1, 7 - r, jnp.where(q == 2, r - 8, 15 - r)))
    return 4 * z + q


def _neighbor_barrier(left, right):
    barrier = pltpu.get_barrier_semaphore()
    for nbr in (left, right):
        pl.semaphore_signal(
            barrier, inc=1, device_id=(nbr,),
            device_id_type=pl.DeviceIdType.MESH,
        )
    pl.semaphore_wait(barrier, 2)


def _prep_body(xf_ref, wdkv_ref, wuk_ref, wuv_ref, wq_ref, wqr_ref, wkr_ref,
               kcat_ref, v_ref, qcat_ref,
               c_all, wstage, wrecv, kvacc,
               a2a_send_sems, a2a_recv_sems,
               cw_send_sems, cw_recv_sems, ccw_send_sems, ccw_recv_sems):
    my_mesh = lax.axis_index("i")
    my = _ring_pos(my_mesh)
    left = _mesh_of(my - 1)
    right = _mesh_of(my + 1)
    bf = jnp.bfloat16

    barrier = pltpu.get_barrier_semaphore()
    for off in range(1, N_DEV):
        pl.semaphore_signal(
            barrier, inc=1, device_id=((my_mesh + off) % N_DEV,),
            device_id_type=pl.DeviceIdType.MESH,
        )
    pl.semaphore_wait(barrier, N_DEV - 1)

    for d in range(N_DEV):
        wstage[d, 0] = wuk_ref[:, d * DH:(d + 1) * DH]
        wstage[d, 1] = wuv_ref[:, d * DH:(d + 1) * DH]
    wrecv[my_mesh] = wstage[my_mesh]
    a2a = []
    for off in range(1, N_DEV):
        dst = (my_mesh + off) % N_DEV
        r = pltpu.make_async_remote_copy(
            src_ref=wstage.at[dst],
            dst_ref=wrecv.at[my_mesh],
            send_sem=a2a_send_sems.at[dst],
            recv_sem=a2a_recv_sems.at[my_mesh],
            device_id=(dst,),
            device_id_type=pl.DeviceIdType.MESH,
        )
        r.start()
        a2a.append(r)

    c_all[my_mesh] = jnp.dot(
        xf_ref[...], wdkv_ref[...],
        preferred_element_type=jnp.float32).astype(bf)
    kvacc[0] = jnp.dot(c_all[my_mesh], wrecv[my_mesh, 0],
                       preferred_element_type=jnp.float32)
    kvacc[1] = jnp.dot(c_all[my_mesh], wrecv[my_mesh, 1],
                       preferred_element_type=jnp.float32)

    def absorb(k):
        w_arrived = pltpu.make_async_remote_copy(
            src_ref=wstage.at[k],
            dst_ref=wrecv.at[k],
            send_sem=a2a_send_sems.at[k],
            recv_sem=a2a_recv_sems.at[k],
            device_id=(k,),
            device_id_type=pl.DeviceIdType.MESH,
        )
        w_arrived.wait_recv()
        kvacc[0] = kvacc[0] + jnp.dot(
            c_all[k], wrecv[k, 0], preferred_element_type=jnp.float32)
        kvacc[1] = kvacc[1] + jnp.dot(
            c_all[k], wrecv[k, 1], preferred_element_type=jnp.float32)

    for s in range(8):
        k_cw_send = _mesh_of(my - s)
        k_cw_recv = _mesh_of(my - s - 1)
        cw = pltpu.make_async_remote_copy(
            src_ref=c_all.at[k_cw_send],
            dst_ref=c_all.at[k_cw_send],
            send_sem=cw_send_sems.at[s],
            recv_sem=cw_recv_sems.at[s],
            device_id=(right,),
            device_id_type=pl.DeviceIdType.MESH,
        )
        cw.start()
        if s < 7:
            k_ccw_send = _mesh_of(my + s)
            k_ccw_recv = _mesh_of(my + s + 1)
            ccw = pltpu.make_async_remote_copy(
                src_ref=c_all.at[k_ccw_send],
                dst_ref=c_all.at[k_ccw_send],
                send_sem=ccw_send_sems.at[s],
                recv_sem=ccw_recv_sems.at[s],
                device_id=(left,),
                device_id_type=pl.DeviceIdType.MESH,
            )
            ccw.start()
        if s >= 1:
            absorb(_mesh_of(my - s))
            absorb(_mesh_of(my + s))
        if s == 0:
            qcat_ref[:, :DH] = jnp.dot(
                xf_ref[...], wq_ref[...],
                preferred_element_type=jnp.float32).astype(bf)
        if s == 1:
            qcat_ref[:, DH:] = jnp.dot(
                xf_ref[...], wqr_ref[...],
                preferred_element_type=jnp.float32).astype(bf)
            kcat_ref[:, DH:] = jnp.dot(
                xf_ref[...], wkr_ref[...],
                preferred_element_type=jnp.float32).astype(bf)
        cw_recv = pltpu.make_async_remote_copy(
            src_ref=c_all.at[k_cw_recv],
            dst_ref=c_all.at[k_cw_recv],
            send_sem=cw_send_sems.at[s],
            recv_sem=cw_recv_sems.at[s],
            device_id=(left,),
            device_id_type=pl.DeviceIdType.MESH,
        )
        cw_recv.wait_recv()
        cw.wait_send()
        if s < 7:
            ccw_recv = pltpu.make_async_remote_copy(
                src_ref=c_all.at[k_ccw_recv],
                dst_ref=c_all.at[k_ccw_recv],
                send_sem=ccw_send_sems.at[s],
                recv_sem=ccw_recv_sems.at[s],
                device_id=(right,),
                device_id_type=pl.DeviceIdType.MESH,
            )
            ccw_recv.wait_recv()
            ccw.wait_send()
    absorb(_mesh_of(my - 8))
    kcat_ref[:, :DH] = kvacc[0].astype(bf)
    v_ref[...] = kvacc[1].astype(bf)
    for r in a2a:
        r.wait_send()


def _prepare(xf, wdkv, wuk, wuv, wq_h, wqr_h, wkr):
    return pl.pallas_call(
        _prep_body,
        out_shape=(
            jax.ShapeDtypeStruct((S, DH + DR), jnp.bfloat16),
            jax.ShapeDtypeStruct((S, DH), jnp.bfloat16),
            jax.ShapeDtypeStruct((S, DH + DR), jnp.bfloat16),
        ),
        in_specs=[pl.BlockSpec(memory_space=pltpu.VMEM)] * 7,
        out_specs=(pl.BlockSpec(memory_space=pltpu.VMEM),) * 3,
        scratch_shapes=[
            pltpu.VMEM((N_DEV, S, DC), jnp.bfloat16),
            pltpu.VMEM((N_DEV, 2, DC, DH), jnp.bfloat16),
            pltpu.VMEM((N_DEV, 2, DC, DH), jnp.bfloat16),
            pltpu.VMEM((2, S, DH), jnp.float32),
            pltpu.SemaphoreType.DMA((N_DEV,)),
            pltpu.SemaphoreType.DMA((N_DEV,)),
            pltpu.SemaphoreType.DMA((8,)),
            pltpu.SemaphoreType.DMA((8,)),
            pltpu.SemaphoreType.DMA((7,)),
            pltpu.SemaphoreType.DMA((7,)),
        ],
        compiler_params=pltpu.CompilerParams(collective_id=0),
    )(xf, wdkv, wuk, wuv, wq_h, wqr_h, wkr)


def _ag_body(o_ref, wo_ref, out_ref, slots,
             cw_send_sems, cw_recv_sems, ccw_send_sems, ccw_recv_sems):
    my_mesh = lax.axis_index("i")
    my = _ring_pos(my_mesh)
    left = _mesh_of(my - 1)
    right = _mesh_of(my + 1)
    _neighbor_barrier(left, right)

    slots[my_mesh] = o_ref[...]

    def head_out(k):
        return jnp.dot(
            slots[k], wo_ref[pl.ds(k * DH, DH), :],
            preferred_element_type=jnp.float32,
        )

    for s in range(8):
        k_cw_send = _mesh_of(my - s)
        k_cw_recv = _mesh_of(my - s - 1)
        cw = pltpu.make_async_remote_copy(
            src_ref=slots.at[k_cw_send],
            dst_ref=slots.at[k_cw_send],
            send_sem=cw_send_sems.at[s],
            recv_sem=cw_recv_sems.at[s],
            device_id=(right,),
            device_id_type=pl.DeviceIdType.MESH,
        )
        cw.start()
        if s < 7:
            k_ccw_send = _mesh_of(my + s)
            k_ccw_recv = _mesh_of(my + s + 1)
            ccw = pltpu.make_async_remote_copy(
                src_ref=slots.at[k_ccw_send],
                dst_ref=slots.at[k_ccw_send],
                send_sem=ccw_send_sems.at[s],
                recv_sem=ccw_recv_sems.at[s],
                device_id=(left,),
                device_id_type=pl.DeviceIdType.MESH,
            )
            ccw.start()
        if s == 0:
            out_ref[...] = head_out(my_mesh)
        else:
            out_ref[...] = out_ref[...] + (
                head_out(_mesh_of(my - s)) + head_out(_mesh_of(my + s)))
        cw_recv = pltpu.make_async_remote_copy(
            src_ref=slots.at[k_cw_recv],
            dst_ref=slots.at[k_cw_recv],
            send_sem=cw_send_sems.at[s],
            recv_sem=cw_recv_sems.at[s],
            device_id=(left,),
            device_id_type=pl.DeviceIdType.MESH,
        )
        cw_recv.wait_recv()
        cw.wait_send()
        if s < 7:
            ccw_recv = pltpu.make_async_remote_copy(
                src_ref=slots.at[k_ccw_recv],
                dst_ref=slots.at[k_ccw_recv],
                send_sem=ccw_send_sems.at[s],
                recv_sem=ccw_recv_sems.at[s],
                device_id=(right,),
                device_id_type=pl.DeviceIdType.MESH,
            )
            ccw_recv.wait_recv()
            ccw.wait_send()
    out_ref[...] = out_ref[...] + head_out(_mesh_of(my - 8))


def _all_gather_matmul(o_h, wo):
    return pl.pallas_call(
        _ag_body,
        out_shape=jax.ShapeDtypeStruct((S, D), jnp.float32),
        in_specs=[
            pl.BlockSpec(memory_space=pltpu.VMEM),
            pl.BlockSpec(memory_space=pltpu.VMEM),
        ],
        out_specs=pl.BlockSpec(memory_space=pltpu.VMEM),
        scratch_shapes=[
            pltpu.VMEM((N_DEV, S, DH), jnp.bfloat16),
            pltpu.SemaphoreType.DMA((8,)),
            pltpu.SemaphoreType.DMA((8,)),
            pltpu.SemaphoreType.DMA((7,)),
            pltpu.SemaphoreType.DMA((7,)),
        ],
        compiler_params=pltpu.CompilerParams(collective_id=1),
    )(o_h, wo)


def kernel(x, Wdkv, Wuk, Wuv, Wq, Wqr, Wkr, Wo):
    my = lax.axis_index("i")
    bf = jnp.bfloat16
    xf = x[0].astype(bf)
    Wq_h = lax.dynamic_slice(Wq, (0, my * DH), (D, DH)).astype(bf)
    Wqr_h = lax.dynamic_slice(Wqr, (0, my * DR), (D, DR)).astype(bf)
    Kcat, V_h, Qcat = _prepare(
        xf, Wdkv.astype(bf), Wuk.astype(bf), Wuv.astype(bf),
        Wq_h, Wqr_h, Wkr.astype(bf))

    scale = (DH + DR) ** -0.5
    scores = jnp.dot(Qcat, Kcat.T, preferred_element_type=jnp.float32) * scale
    p = jnp.exp(scores)
    p = p / jnp.sum(p, axis=-1, keepdims=True)
    O_h = jnp.dot(p.astype(bf), V_h)

    out = _all_gather_matmul(O_h, Wo.astype(bf))
    return out[None]


# device time: 100218 ns/iter; 1.2599x vs baseline; 1.1347x over previous
import jax
import jax.numpy as jnp
from jax import lax
from jax.experimental import pallas as pl
from jax.experimental.pallas import tpu as pltpu

N_DEV = 16
H = 16
DH = 128
DR = 32
DC = 128
S = 1024
D = 2048
HS = S // 2


def _halfpipe_ag(buf, my, left, right,
                 cw_send_sems, cw_recv_sems, ccw_send_sems, ccw_recv_sems,
                 process):
    def sub(k, h):
        return buf.at[k, pl.ds(h * HS, HS)]

    def mk(k, h, sems_s, sems_r, m, dev):
        return pltpu.make_async_remote_copy(
            src_ref=sub(k, h),
            dst_ref=sub(k, h),
            send_sem=sems_s.at[m],
            recv_sem=sems_r.at[m],
            device_id=(dev,),
            device_id_type=pl.DeviceIdType.MESH,
        )

    sends = []
    for m in range(16):
        s, h = m // 2, m % 2
        if m >= 2:
            mk(_mesh_of(my - s), h, cw_send_sems, cw_recv_sems,
               m - 2, left).wait_recv()
            if m < 14:
                mk(_mesh_of(my + s), h, ccw_send_sems, ccw_recv_sems,
                   m - 2, right).wait_recv()
        cw = mk(_mesh_of(my - s), h, cw_send_sems, cw_recv_sems, m, right)
        cw.start()
        sends.append(cw)
        if m < 14:
            ccw = mk(_mesh_of(my + s), h, ccw_send_sems, ccw_recv_sems,
                     m, left)
            ccw.start()
            sends.append(ccw)
        if m == 1:
            process("own", 0)
        elif h == 1:
            process("cw", s)
            if s <= 6:
                process("ccw", s)
    mk(_mesh_of(my - 8), 0, cw_send_sems, cw_recv_sems, 14, left).wait_recv()
    mk(_mesh_of(my - 8), 1, cw_send_sems, cw_recv_sems, 15, left).wait_recv()
    mk(_mesh_of(my + 7), 0, ccw_send_sems, ccw_recv_sems, 12,
       right).wait_recv()
    mk(_mesh_of(my + 7), 1, ccw_send_sems, ccw_recv_sems, 13,
       right).wait_recv()
    process("cw", 8)
    process("ccw", 7)
    for r in sends:
        r.wait_send()


def _ring_pos(mesh):
    q = mesh % 4
    z = mesh // 4
    return jnp.where(
        q == 0, z,
        jnp.where(q == 1, 7 - z, jnp.where(q == 2, 8 + z, 15 - z)))


def _mesh_of(r):
    r = r % N_DEV
    q = r // 4
    z = jnp.where(
        q == 0, r,
        jnp.where(q == 1, 7 - r, jnp.where(q == 2, r - 8, 15 - r)))
    return 4 * z + q


def _neighbor_barrier(left, right):
    barrier = pltpu.get_barrier_semaphore()
    for nbr in (left, right):
        pl.semaphore_signal(
            barrier, inc=1, device_id=(nbr,),
            device_id_type=pl.DeviceIdType.MESH,
        )
    pl.semaphore_wait(barrier, 2)


def _prep_body(xf_ref, wdkv_ref, wuk_ref, wuv_ref, wq_ref, wqr_ref, wkr_ref,
               kcat_ref, v_ref, qcat_ref,
               c_all, wstage, wrecv, kvacc,
               a2a_send_sems, a2a_recv_sems,
               cw_send_sems, cw_recv_sems, ccw_send_sems, ccw_recv_sems):
    my_mesh = lax.axis_index("i")
    my = _ring_pos(my_mesh)
    left = _mesh_of(my - 1)
    right = _mesh_of(my + 1)
    bf = jnp.bfloat16

    barrier = pltpu.get_barrier_semaphore()
    for off in range(1, N_DEV):
        pl.semaphore_signal(
            barrier, inc=1, device_id=((my_mesh + off) % N_DEV,),
            device_id_type=pl.DeviceIdType.MESH,
        )
    pl.semaphore_wait(barrier, N_DEV - 1)

    for d in range(N_DEV):
        wstage[d, 0] = wuk_ref[:, d * DH:(d + 1) * DH]
        wstage[d, 1] = wuv_ref[:, d * DH:(d + 1) * DH]
    wrecv[my_mesh] = wstage[my_mesh]
    a2a = []
    for off in range(1, N_DEV):
        dst = (my_mesh + off) % N_DEV
        r = pltpu.make_async_remote_copy(
            src_ref=wstage.at[dst],
            dst_ref=wrecv.at[my_mesh],
            send_sem=a2a_send_sems.at[dst],
            recv_sem=a2a_recv_sems.at[my_mesh],
            device_id=(dst,),
            device_id_type=pl.DeviceIdType.MESH,
        )
        r.start()
        a2a.append(r)

    c_all[my_mesh] = jnp.dot(
        xf_ref[...], wdkv_ref[...],
        preferred_element_type=jnp.float32).astype(bf)

    def absorb(k):
        w_arrived = pltpu.make_async_remote_copy(
            src_ref=wstage.at[k],
            dst_ref=wrecv.at[k],
            send_sem=a2a_send_sems.at[k],
            recv_sem=a2a_recv_sems.at[k],
            device_id=(k,),
            device_id_type=pl.DeviceIdType.MESH,
        )
        w_arrived.wait_recv()
        kvacc[0] = kvacc[0] + jnp.dot(
            c_all[k], wrecv[k, 0], preferred_element_type=jnp.float32)
        kvacc[1] = kvacc[1] + jnp.dot(
            c_all[k], wrecv[k, 1], preferred_element_type=jnp.float32)

    def process(kind, s):
        if kind == "own":
            kvacc[0] = jnp.dot(c_all[my_mesh], wrecv[my_mesh, 0],
                               preferred_element_type=jnp.float32)
            kvacc[1] = jnp.dot(c_all[my_mesh], wrecv[my_mesh, 1],
                               preferred_element_type=jnp.float32)
            qcat_ref[:, :DH] = jnp.dot(
                xf_ref[...], wq_ref[...],
                preferred_element_type=jnp.float32).astype(bf)
            return
        absorb(_mesh_of(my - s) if kind == "cw" else _mesh_of(my + s))
        if kind == "cw" and s == 1:
            qcat_ref[:, DH:] = jnp.dot(
                xf_ref[...], wqr_ref[...],
                preferred_element_type=jnp.float32).astype(bf)
            kcat_ref[:, DH:] = jnp.dot(
                xf_ref[...], wkr_ref[...],
                preferred_element_type=jnp.float32).astype(bf)

    _halfpipe_ag(c_all, my, left, right,
                 cw_send_sems, cw_recv_sems, ccw_send_sems, ccw_recv_sems,
                 process)
    kcat_ref[:, :DH] = kvacc[0].astype(bf)
    v_ref[...] = kvacc[1].astype(bf)
    for r in a2a:
        r.wait_send()


def _prepare(xf, wdkv, wuk, wuv, wq_h, wqr_h, wkr):
    return pl.pallas_call(
        _prep_body,
        out_shape=(
            jax.ShapeDtypeStruct((S, DH + DR), jnp.bfloat16),
            jax.ShapeDtypeStruct((S, DH), jnp.bfloat16),
            jax.ShapeDtypeStruct((S, DH + DR), jnp.bfloat16),
        ),
        in_specs=[pl.BlockSpec(memory_space=pltpu.VMEM)] * 7,
        out_specs=(pl.BlockSpec(memory_space=pltpu.VMEM),) * 3,
        scratch_shapes=[
            pltpu.VMEM((N_DEV, S, DC), jnp.bfloat16),
            pltpu.VMEM((N_DEV, 2, DC, DH), jnp.bfloat16),
            pltpu.VMEM((N_DEV, 2, DC, DH), jnp.bfloat16),
            pltpu.VMEM((2, S, DH), jnp.float32),
            pltpu.SemaphoreType.DMA((N_DEV,)),
            pltpu.SemaphoreType.DMA((N_DEV,)),
            pltpu.SemaphoreType.DMA((16,)),
            pltpu.SemaphoreType.DMA((16,)),
            pltpu.SemaphoreType.DMA((14,)),
            pltpu.SemaphoreType.DMA((14,)),
        ],
        compiler_params=pltpu.CompilerParams(collective_id=0),
    )(xf, wdkv, wuk, wuv, wq_h, wqr_h, wkr)


def _ag_body(o_ref, wo_ref, out_ref, slots,
             cw_send_sems, cw_recv_sems, ccw_send_sems, ccw_recv_sems):
    my_mesh = lax.axis_index("i")
    my = _ring_pos(my_mesh)
    left = _mesh_of(my - 1)
    right = _mesh_of(my + 1)
    _neighbor_barrier(left, right)

    slots[my_mesh] = o_ref[...]

    def head_out(k):
        return jnp.dot(
            slots[k], wo_ref[pl.ds(k * DH, DH), :],
            preferred_element_type=jnp.float32,
        )

    def process(kind, s):
        if kind == "own":
            out_ref[...] = head_out(my_mesh)
        else:
            k = _mesh_of(my - s) if kind == "cw" else _mesh_of(my + s)
            out_ref[...] = out_ref[...] + head_out(k)

    _halfpipe_ag(slots, my, left, right,
                 cw_send_sems, cw_recv_sems, ccw_send_sems, ccw_recv_sems,
                 process)


def _all_gather_matmul(o_h, wo):
    return pl.pallas_call(
        _ag_body,
        out_shape=jax.ShapeDtypeStruct((S, D), jnp.float32),
        in_specs=[
            pl.BlockSpec(memory_space=pltpu.VMEM),
            pl.BlockSpec(memory_space=pltpu.VMEM),
        ],
        out_specs=pl.BlockSpec(memory_space=pltpu.VMEM),
        scratch_shapes=[
            pltpu.VMEM((N_DEV, S, DH), jnp.bfloat16),
            pltpu.SemaphoreType.DMA((16,)),
            pltpu.SemaphoreType.DMA((16,)),
            pltpu.SemaphoreType.DMA((14,)),
            pltpu.SemaphoreType.DMA((14,)),
        ],
        compiler_params=pltpu.CompilerParams(collective_id=1),
    )(o_h, wo)


def kernel(x, Wdkv, Wuk, Wuv, Wq, Wqr, Wkr, Wo):
    my = lax.axis_index("i")
    bf = jnp.bfloat16
    xf = x[0].astype(bf)
    Wq_h = lax.dynamic_slice(Wq, (0, my * DH), (D, DH)).astype(bf)
    Wqr_h = lax.dynamic_slice(Wqr, (0, my * DR), (D, DR)).astype(bf)
    Kcat, V_h, Qcat = _prepare(
        xf, Wdkv.astype(bf), Wuk.astype(bf), Wuv.astype(bf),
        Wq_h, Wqr_h, Wkr.astype(bf))

    scale = (DH + DR) ** -0.5
    scores = jnp.dot(Qcat, Kcat.T, preferred_element_type=jnp.float32) * scale
    p = jnp.exp(scores)
    p = p / jnp.sum(p, axis=-1, keepdims=True)
    O_h = jnp.dot(p.astype(bf), V_h)

    out = _all_gather_matmul(O_h, Wo.astype(bf))
    return out[None]


# device time: 99623 ns/iter; 1.2675x vs baseline; 1.0060x over previous
import jax
import jax.numpy as jnp
from jax import lax
from jax.experimental import pallas as pl
from jax.experimental.pallas import tpu as pltpu

N_DEV = 16
H = 16
DH = 128
DR = 32
DC = 128
S = 1024
D = 2048
F = 4
HS = S // F


def _halfpipe_ag(buf, my, left, right,
                 cw_send_sems, cw_recv_sems, ccw_send_sems, ccw_recv_sems,
                 process):
    def sub(k, h):
        return buf.at[k, pl.ds(h * HS, HS)]

    def mk(k, h, sems_s, sems_r, m, dev):
        return pltpu.make_async_remote_copy(
            src_ref=sub(k, h),
            dst_ref=sub(k, h),
            send_sem=sems_s.at[m],
            recv_sem=sems_r.at[m],
            device_id=(dev,),
            device_id_type=pl.DeviceIdType.MESH,
        )

    sends = []
    for m in range(8 * F):
        s, h = m // F, m % F
        if m >= F:
            mk(_mesh_of(my - s), h, cw_send_sems, cw_recv_sems,
               m - F, left).wait_recv()
            if m < 7 * F:
                mk(_mesh_of(my + s), h, ccw_send_sems, ccw_recv_sems,
                   m - F, right).wait_recv()
        cw = mk(_mesh_of(my - s), h, cw_send_sems, cw_recv_sems, m, right)
        cw.start()
        sends.append(cw)
        if m < 7 * F:
            ccw = mk(_mesh_of(my + s), h, ccw_send_sems, ccw_recv_sems,
                     m, left)
            ccw.start()
            sends.append(ccw)
        if m == F - 1:
            process("own", 0)
        elif h == F - 1:
            process("cw", s)
            if s <= 6:
                process("ccw", s)
    for h in range(F):
        mk(_mesh_of(my - 8), h, cw_send_sems, cw_recv_sems,
           7 * F + h, left).wait_recv()
        mk(_mesh_of(my + 7), h, ccw_send_sems, ccw_recv_sems,
           6 * F + h, right).wait_recv()
    process("cw", 8)
    process("ccw", 7)
    for r in sends:
        r.wait_send()


def _ring_pos(mesh):
    q = mesh % 4
    z = mesh // 4
    return jnp.where(
        q == 0, z,
        jnp.where(q == 1, 7 - z, jnp.where(q == 2, 8 + z, 15 - z)))


def _mesh_of(r):
    r = r % N_DEV
    q = r // 4
    z = jnp.where(
        q == 0, r,
        jnp.where(q == 1, 7 - r, jnp.where(q == 2, r - 8, 15 - r)))
    return 4 * z + q


def _neighbor_barrier(left, right):
    barrier = pltpu.get_barrier_semaphore()
    for nbr in (left, right):
        pl.semaphore_signal(
            barrier, inc=1, device_id=(nbr,),
            device_id_type=pl.DeviceIdType.MESH,
        )
    pl.semaphore_wait(barrier, 2)


def _prep_body(xf_ref, wdkv_ref, wuk_ref, wuv_ref, wq_ref, wqr_ref, wkr_ref,
               kcat_ref, v_ref, qcat_ref,
               c_all, wstage, wrecv, kvacc,
               a2a_send_sems, a2a_recv_sems,
               cw_send_sems, cw_recv_sems, ccw_send_sems, ccw_recv_sems):
    my_mesh = lax.axis_index("i")
    my = _ring_pos(my_mesh)
    left = _mesh_of(my - 1)
    right = _mesh_of(my + 1)
    bf = jnp.bfloat16

    barrier = pltpu.get_barrier_semaphore()
    for off in range(1, N_DEV):
        pl.semaphore_signal(
            barrier, inc=1, device_id=((my_mesh + off) % N_DEV,),
            device_id_type=pl.DeviceIdType.MESH,
        )
    pl.semaphore_wait(barrier, N_DEV - 1)

    for d in range(N_DEV):
        wstage[d, 0] = wuk_ref[:, d * DH:(d + 1) * DH]
        wstage[d, 1] = wuv_ref[:, d * DH:(d + 1) * DH]
    wrecv[my_mesh] = wstage[my_mesh]
    a2a = []
    for off in range(1, N_DEV):
        dst = (my_mesh + off) % N_DEV
        r = pltpu.make_async_remote_copy(
            src_ref=wstage.at[dst],
            dst_ref=wrecv.at[my_mesh],
            send_sem=a2a_send_sems.at[dst],
            recv_sem=a2a_recv_sems.at[my_mesh],
            device_id=(dst,),
            device_id_type=pl.DeviceIdType.MESH,
        )
        r.start()
        a2a.append(r)

    c_all[my_mesh] = jnp.dot(
        xf_ref[...], wdkv_ref[...],
        preferred_element_type=jnp.float32).astype(bf)

    def absorb(k):
        w_arrived = pltpu.make_async_remote_copy(
            src_ref=wstage.at[k],
            dst_ref=wrecv.at[k],
            send_sem=a2a_send_sems.at[k],
            recv_sem=a2a_recv_sems.at[k],
            device_id=(k,),
            device_id_type=pl.DeviceIdType.MESH,
        )
        w_arrived.wait_recv()
        kvacc[0] = kvacc[0] + jnp.dot(
            c_all[k], wrecv[k, 0], preferred_element_type=jnp.float32)
        kvacc[1] = kvacc[1] + jnp.dot(
            c_all[k], wrecv[k, 1], preferred_element_type=jnp.float32)

    def process(kind, s):
        if kind == "own":
            kvacc[0] = jnp.dot(c_all[my_mesh], wrecv[my_mesh, 0],
                               preferred_element_type=jnp.float32)
            kvacc[1] = jnp.dot(c_all[my_mesh], wrecv[my_mesh, 1],
                               preferred_element_type=jnp.float32)
            qcat_ref[:, :DH] = jnp.dot(
                xf_ref[...], wq_ref[...],
                preferred_element_type=jnp.float32).astype(bf)
            return
        absorb(_mesh_of(my - s) if kind == "cw" else _mesh_of(my + s))
        if kind == "cw" and s == 1:
            qcat_ref[:, DH:] = jnp.dot(
                xf_ref[...], wqr_ref[...],
                preferred_element_type=jnp.float32).astype(bf)
            kcat_ref[:, DH:] = jnp.dot(
                xf_ref[...], wkr_ref[...],
                preferred_element_type=jnp.float32).astype(bf)

    _halfpipe_ag(c_all, my, left, right,
                 cw_send_sems, cw_recv_sems, ccw_send_sems, ccw_recv_sems,
                 process)
    kcat_ref[:, :DH] = kvacc[0].astype(bf)
    v_ref[...] = kvacc[1].astype(bf)
    for r in a2a:
        r.wait_send()


def _prepare(xf, wdkv, wuk, wuv, wq_h, wqr_h, wkr):
    return pl.pallas_call(
        _prep_body,
        out_shape=(
            jax.ShapeDtypeStruct((S, DH + DR), jnp.bfloat16),
            jax.ShapeDtypeStruct((S, DH), jnp.bfloat16),
            jax.ShapeDtypeStruct((S, DH + DR), jnp.bfloat16),
        ),
        in_specs=[pl.BlockSpec(memory_space=pltpu.VMEM)] * 7,
        out_specs=(pl.BlockSpec(memory_space=pltpu.VMEM),) * 3,
        scratch_shapes=[
            pltpu.VMEM((N_DEV, S, DC), jnp.bfloat16),
            pltpu.VMEM((N_DEV, 2, DC, DH), jnp.bfloat16),
            pltpu.VMEM((N_DEV, 2, DC, DH), jnp.bfloat16),
            pltpu.VMEM((2, S, DH), jnp.float32),
            pltpu.SemaphoreType.DMA((N_DEV,)),
            pltpu.SemaphoreType.DMA((N_DEV,)),
            pltpu.SemaphoreType.DMA((8 * F,)),
            pltpu.SemaphoreType.DMA((8 * F,)),
            pltpu.SemaphoreType.DMA((7 * F,)),
            pltpu.SemaphoreType.DMA((7 * F,)),
        ],
        compiler_params=pltpu.CompilerParams(collective_id=0),
    )(xf, wdkv, wuk, wuv, wq_h, wqr_h, wkr)


def _ag_body(o_ref, wo_ref, out_ref, slots,
             cw_send_sems, cw_recv_sems, ccw_send_sems, ccw_recv_sems):
    my_mesh = lax.axis_index("i")
    my = _ring_pos(my_mesh)
    left = _mesh_of(my - 1)
    right = _mesh_of(my + 1)
    _neighbor_barrier(left, right)

    slots[my_mesh] = o_ref[...]

    def head_out(k):
        return jnp.dot(
            slots[k], wo_ref[pl.ds(k * DH, DH), :],
            preferred_element_type=jnp.float32,
        )

    def process(kind, s):
        if kind == "own":
            out_ref[...] = head_out(my_mesh)
        else:
            k = _mesh_of(my - s) if kind == "cw" else _mesh_of(my + s)
            out_ref[...] = out_ref[...] + head_out(k)

    _halfpipe_ag(slots, my, left, right,
                 cw_send_sems, cw_recv_sems, ccw_send_sems, ccw_recv_sems,
                 process)


def _all_gather_matmul(o_h, wo):
    return pl.pallas_call(
        _ag_body,
        out_shape=jax.ShapeDtypeStruct((S, D), jnp.float32),
        in_specs=[
            pl.BlockSpec(memory_space=pltpu.VMEM),
            pl.BlockSpec(memory_space=pltpu.VMEM),
        ],
        out_specs=pl.BlockSpec(memory_space=pltpu.VMEM),
        scratch_shapes=[
            pltpu.VMEM((N_DEV, S, DH), jnp.bfloat16),
            pltpu.SemaphoreType.DMA((8 * F,)),
            pltpu.SemaphoreType.DMA((8 * F,)),
            pltpu.SemaphoreType.DMA((7 * F,)),
            pltpu.SemaphoreType.DMA((7 * F,)),
        ],
        compiler_params=pltpu.CompilerParams(collective_id=1),
    )(o_h, wo)


def kernel(x, Wdkv, Wuk, Wuv, Wq, Wqr, Wkr, Wo):
    my = lax.axis_index("i")
    bf = jnp.bfloat16
    xf = x[0].astype(bf)
    Wq_h = lax.dynamic_slice(Wq, (0, my * DH), (D, DH)).astype(bf)
    Wqr_h = lax.dynamic_slice(Wqr, (0, my * DR), (D, DR)).astype(bf)
    Kcat, V_h, Qcat = _prepare(
        xf, Wdkv.astype(bf), Wuk.astype(bf), Wuv.astype(bf),
        Wq_h, Wqr_h, Wkr.astype(bf))

    scale = (DH + DR) ** -0.5
    scores = jnp.dot(Qcat, Kcat.T, preferred_element_type=jnp.float32) * scale
    p = jnp.exp(scores)
    p = p / jnp.sum(p, axis=-1, keepdims=True)
    O_h = jnp.dot(p.astype(bf), V_h)

    out = _all_gather_matmul(O_h, Wo.astype(bf))
    return out[None]
